# speedup vs baseline: 8.3910x; 8.3910x over previous
import functools

import jax
import jax.numpy as jnp
from jax import lax
from jax.experimental import pallas as pl
from jax.experimental.pallas import tpu as pltpu
from jax.experimental.pallas import tpu_sc as plsc

NUM_ITEMS = 100000
EMBED_DIM = 128
BATCH = 4096
HIST_LEN = 50

NC, NS, LANES = 2, 16, 16
NW = NC * NS
SPW = BATCH // NW
SGRP = 4
GIDX = SGRP * HIST_LEN
NG = SPW // SGRP
SPLIT = 104
NCH = EMBED_DIM // LANES

_MESH = plsc.VectorSubcoreMesh(
    core_axis_name="c", subcore_axis_name="s", num_cores=NC, num_subcores=NS
)


@functools.partial(
    pl.kernel,
    out_type=jax.ShapeDtypeStruct((BATCH, EMBED_DIM), jnp.float32),
    mesh=_MESH,
    scratch_types=[
        pltpu.VMEM((SPW * HIST_LEN,), jnp.int32),
        pltpu.VMEM((GIDX, EMBED_DIM), jnp.float32),
        pltpu.VMEM((GIDX, EMBED_DIM), jnp.float32),
        pltpu.VMEM((SPW, EMBED_DIM), jnp.float32),
        pltpu.SemaphoreType.DMA,
        pltpu.SemaphoreType.DMA,
    ],
)
def _session_mean_sc(idx_hbm, table_hbm, out_hbm, idx_v, rows0, rows1, out_v, sem0, sem1):
    wid = lax.axis_index("s") * NC + lax.axis_index("c")
    base = pl.multiple_of(wid * (SPW * HIST_LEN), 8)
    pltpu.sync_copy(idx_hbm.at[pl.ds(base, SPW * HIST_LEN)], idx_v)

    def issue(g, buf, sem):
        off = pl.multiple_of(g * GIDX, 8)
        pltpu.async_copy(
            table_hbm.at[idx_v.at[pl.ds(off, SPLIT)]], buf.at[pl.ds(0, SPLIT)], sem
        )
        pltpu.async_copy(
            table_hbm.at[idx_v.at[pl.ds(off + SPLIT, GIDX - SPLIT)]],
            buf.at[pl.ds(SPLIT, GIDX - SPLIT)],
            sem,
        )

    def wait(g, buf, sem):
        off = pl.multiple_of(g * GIDX, 8)
        pltpu.make_async_copy(
            table_hbm.at[idx_v.at[pl.ds(off, SPLIT)]], buf.at[pl.ds(0, SPLIT)], sem
        ).wait()
        pltpu.make_async_copy(
            table_hbm.at[idx_v.at[pl.ds(off + SPLIT, GIDX - SPLIT)]],
            buf.at[pl.ds(SPLIT, GIDX - SPLIT)],
            sem,
        ).wait()

    inv_l = jnp.float32(1.0 / HIST_LEN)

    def accumulate(g, buf):
        for s in range(SGRP):
            r0 = s * HIST_LEN

            def jbody(j, accs, _r0=r0):
                return tuple(
                    accs[c] + buf[_r0 + j, pl.ds(c * LANES, LANES)] for c in range(NCH)
                )

            accs = lax.fori_loop(
                0, HIST_LEN, jbody,
                tuple(jnp.zeros((LANES,), jnp.float32) for _ in range(NCH)),
            )
            orow = g * SGRP + s
            for c in range(NCH):
                out_v[orow, pl.ds(c * LANES, LANES)] = accs[c] * inv_l

    issue(0, rows0, sem0)

    def body(i, carry):
        g0 = i * 2
        wait(g0, rows0, sem0)
        issue(g0 + 1, rows1, sem1)
        accumulate(g0, rows0)
        wait(g0 + 1, rows1, sem1)

        @pl.when(i < NG // 2 - 1)
        def _():
            issue(g0 + 2, rows0, sem0)

        accumulate(g0 + 1, rows1)
        return carry

    lax.fori_loop(0, NG // 2, body, 0)

    obase = pl.multiple_of(wid * SPW, 8)
    pltpu.sync_copy(out_v, out_hbm.at[pl.ds(obase, SPW)])


def kernel(sess2items, pos_items, neg_items, item_embeddings):
    idx_flat = sess2items.astype(jnp.int32).reshape(-1)
    session_embedding = _session_mean_sc(idx_flat, item_embeddings)
    return (session_embedding, item_embeddings)
